# baseline (speedup 1.0000x reference)
import functools

import jax
import jax.numpy as jnp
from jax import lax
from jax.experimental import pallas as pl
from jax.experimental.pallas import tpu as pltpu
from jax.experimental.pallas import tpu_sc as plsc

VOCAB = 100000
D = 300
B = 1024
L = 200
HID = 32
OUT = 2

NC = 2
NS = 16
NW = NC * NS
RPW = B // NW
G = 40
NCHUNK = L // G
NBUF = 4
DP = 304
NACC = DP // 16
V2 = VOCAB // 2
D2 = 2 * D
LP = 208


def _sc_sums(x, table2):
  mesh = plsc.VectorSubcoreMesh(core_axis_name="c", subcore_axis_name="s")

  @functools.partial(
      pl.kernel,
      out_type=jax.ShapeDtypeStruct((B, DP), jnp.float32),
      mesh=mesh,
      compiler_params=pltpu.CompilerParams(
          use_tc_tiling_on_sc=False, needs_layout_passes=False),
      scratch_types=(
          [pltpu.VMEM((RPW, LP), jnp.int32),
           pltpu.VMEM((RPW, LP), jnp.int32)]
          + [pltpu.VMEM((G, D2), jnp.float32) for _ in range(NBUF)]
          + [pltpu.VMEM((RPW, DP), jnp.float32)]
          + [pltpu.SemaphoreType.DMA for _ in range(NBUF)]
      ),
  )
  def k(x_hbm, tab_hbm, out_hbm, idx_v, off_v, *rest):
    bufs = rest[:NBUF]
    ostage = rest[NBUF]
    sems = rest[NBUF + 1:]
    wid = lax.axis_index("s") * NC + lax.axis_index("c")
    base = wid * RPW
    pltpu.sync_copy(x_hbm.at[pl.ds(base, RPW)],
                    idx_v.at[pl.ds(0, RPW), pl.ds(0, L)])

    iota = lax.iota(jnp.int32, 16)
    sixteen = jnp.full((16,), 16, jnp.int32)
    c299 = jnp.full((16,), 299, jnp.int32)

    @pl.loop(0, RPW)
    def _prep(i):
      for j in range(0, LP, 16):
        v = idx_v[i, pl.ds(j, 16)]
        off_v[i, pl.ds(j, 16)] = (v & 1) * D
        idx_v[i, pl.ds(j, 16)] = v >> 1

    def issue(row, c, b):
      pltpu.async_copy(tab_hbm.at[idx_v.at[row, pl.ds(c * G, G)]],
                       bufs[b], sems[b])

    def wait(row, c, b):
      pltpu.make_async_copy(tab_hbm.at[idx_v.at[row, pl.ds(c * G, G)]],
                            bufs[b], sems[b]).wait()

    def mk_body(buf, row, c):
      def body(r, accs):
        row16 = jnp.full((16,), row, jnp.int32)
        pos16 = jnp.full((16,), c * G + r, jnp.int32)
        bse = plsc.load_gather(off_v, [row16, pos16])
        limit = bse + c299
        col = bse + iota
        r16 = jnp.full((16,), r, jnp.int32)
        out = []
        for kk, a in enumerate(accs):
          cc = jnp.minimum(col, limit) if kk == NACC - 1 else col
          out.append(a + plsc.load_gather(buf, [r16, cc]))
          col = col + sixteen
        return tuple(out)
      return body

    for t in range(NBUF):
      issue(0, t, t)

    zeros = tuple(jnp.zeros((16,), jnp.float32) for _ in range(NACC))

    @pl.loop(0, RPW, step=NBUF)
    def _blk(i):
      accs = zeros
      for t in range(NBUF * NCHUNK):
        r_off, c = divmod(t, NCHUNK)
        row = i + r_off
        b = t % NBUF
        if c == 0:
          accs = zeros
        wait(row, c, b)
        accs = lax.fori_loop(0, G, mk_body(bufs[b], row, c), accs)
        r4, c4 = divmod(t + NBUF, NCHUNK)
        nrow = i + r4

        @pl.when(nrow < RPW)
        def _():
          issue(nrow, c4, b)

        if c == NCHUNK - 1:
          for kk in range(NACC):
            ostage[row, pl.ds(kk * 16, 16)] = accs[kk]

    pltpu.sync_copy(ostage, out_hbm.at[pl.ds(base, RPW)])

  return k(x, table2)


def _mlp_body(s_ref, w1_ref, b1_ref, w2_ref, b2_ref, o_ref):
  h = jnp.dot(s_ref[:, pl.ds(0, D)], w1_ref[...],
              preferred_element_type=jnp.float32)
  h = jnp.maximum(h + b1_ref[...], 0.0)
  logits = jnp.dot(h, w2_ref[...], preferred_element_type=jnp.float32)
  logits = logits + b2_ref[...]
  m = jnp.max(logits, axis=0, keepdims=True)
  z = logits - m
  o_ref[...] = z - jnp.log(jnp.sum(jnp.exp(z), axis=0, keepdims=True))


def kernel(x, emb_table, V_w, V_b, W_w, W_b):
  x = x.astype(jnp.int32)
  table2 = emb_table.reshape(V2, D2)
  sums = _sc_sums(x, table2)
  w1 = V_w.T * (1.0 / L)
  b1 = V_b.reshape(1, HID)
  w2 = W_w.T
  b2 = W_b.reshape(1, OUT)
  return pl.pallas_call(
      _mlp_body,
      out_shape=jax.ShapeDtypeStruct((B, OUT), jnp.float32),
  )(sums, w1, b1, w2, b2)


# speedup vs baseline: 1.2189x; 1.2189x over previous
import functools

import jax
import jax.numpy as jnp
from jax import lax
from jax.experimental import pallas as pl
from jax.experimental.pallas import tpu as pltpu
from jax.experimental.pallas import tpu_sc as plsc

VOCAB = 100000
D = 300
B = 1024
L = 200
HID = 32
OUT = 2

NC = 2
NS = 16
NW = NC * NS
RPW = B // NW
G = 40
NCHUNK = L // G
NBUF = 4
DP = 304
NACC = DP // 16
V2 = VOCAB // 2
D2 = 2 * D
LP = 208


def _sc_sums(x, table2):
  mesh = plsc.VectorSubcoreMesh(core_axis_name="c", subcore_axis_name="s")

  @functools.partial(
      pl.kernel,
      out_type=jax.ShapeDtypeStruct((B, DP), jnp.float32),
      mesh=mesh,
      compiler_params=pltpu.CompilerParams(
          use_tc_tiling_on_sc=False, needs_layout_passes=False),
      scratch_types=(
          [pltpu.VMEM((RPW, LP), jnp.int32),
           pltpu.VMEM((RPW, LP), jnp.int32)]
          + [pltpu.VMEM((G, D2), jnp.float32) for _ in range(NBUF)]
          + [pltpu.VMEM((RPW, DP), jnp.float32)]
          + [pltpu.SemaphoreType.DMA for _ in range(NBUF)]
      ),
  )
  def k(x_hbm, tab_hbm, out_hbm, idx_v, off_v, *rest):
    bufs = rest[:NBUF]
    ostage = rest[NBUF]
    sems = rest[NBUF + 1:]
    wid = lax.axis_index("s") * NC + lax.axis_index("c")
    base = wid * RPW
    pltpu.sync_copy(x_hbm.at[pl.ds(base, RPW)],
                    idx_v.at[pl.ds(0, RPW), pl.ds(0, L)])

    iota = lax.iota(jnp.int32, 16)
    sixteen = jnp.full((16,), 16, jnp.int32)
    sixty4 = jnp.full((16,), 64, jnp.int32)
    c299 = jnp.full((16,), 299, jnp.int32)

    @pl.loop(0, RPW)
    def _prep(i):
      for j in range(0, LP, 16):
        v = idx_v[i, pl.ds(j, 16)]
        off_v[i, pl.ds(j, 16)] = (v & 1) * D
        idx_v[i, pl.ds(j, 16)] = v >> 1

    def issue(row, c, b):
      pltpu.async_copy(tab_hbm.at[idx_v.at[row, pl.ds(c * G, G)]],
                       bufs[b], sems[b])

    def wait(row, c, b):
      pltpu.make_async_copy(tab_hbm.at[idx_v.at[row, pl.ds(c * G, G)]],
                            bufs[b], sems[b]).wait()

    def mk_body(buf, row, c):
      def body(r, carry):
        accs, row16 = carry[:NACC], carry[NACC]
        pos16 = jnp.full((16,), c * G + r, jnp.int32)
        bse = plsc.load_gather(off_v, [row16, pos16])
        limit = bse + c299
        c0 = bse + iota
        c1 = c0 + sixteen
        c2 = c1 + sixteen
        c3 = c2 + sixteen
        cols = [c0, c1, c2, c3]
        r16 = jnp.full((16,), r, jnp.int32)
        out = []
        for kk, a in enumerate(accs):
          cc = cols[kk % 4]
          cc = jnp.minimum(cc, limit) if kk == NACC - 1 else cc
          out.append(a + plsc.load_gather(buf, [r16, cc]))
          cols[kk % 4] = cc + sixty4
        return tuple(out) + (row16,)
      return body

    for t in range(NBUF):
      issue(0, t, t)

    zeros = tuple(jnp.zeros((16,), jnp.float32) for _ in range(NACC))

    @pl.loop(0, RPW, step=NBUF)
    def _blk(i):
      accs = zeros
      for t in range(NBUF * NCHUNK):
        r_off, c = divmod(t, NCHUNK)
        row = i + r_off
        b = t % NBUF
        if c == 0:
          accs = zeros
        wait(row, c, b)
        row16 = jnp.full((16,), row, jnp.int32)
        accs = lax.fori_loop(0, G, mk_body(bufs[b], row, c),
                             accs + (row16,))[:NACC]
        r4, c4 = divmod(t + NBUF, NCHUNK)
        nrow = i + r4

        @pl.when(nrow < RPW)
        def _():
          issue(nrow, c4, b)

        if c == NCHUNK - 1:
          for kk in range(NACC):
            ostage[row, pl.ds(kk * 16, 16)] = accs[kk]

    pltpu.sync_copy(ostage, out_hbm.at[pl.ds(base, RPW)])

  return k(x, table2)


def _mlp_body(s_ref, w1_ref, b1_ref, w2_ref, b2_ref, o_ref):
  h = jnp.dot(s_ref[:, pl.ds(0, D)], w1_ref[...],
              preferred_element_type=jnp.float32)
  h = jnp.maximum(h + b1_ref[...], 0.0)
  logits = jnp.dot(h, w2_ref[...], preferred_element_type=jnp.float32)
  logits = logits + b2_ref[...]
  m = jnp.max(logits, axis=0, keepdims=True)
  z = logits - m
  o_ref[...] = z - jnp.log(jnp.sum(jnp.exp(z), axis=0, keepdims=True))


def kernel(x, emb_table, V_w, V_b, W_w, W_b):
  x = x.astype(jnp.int32)
  table2 = emb_table.reshape(V2, D2)
  sums = _sc_sums(x, table2)
  w1 = V_w.T * (1.0 / L)
  b1 = V_b.reshape(1, HID)
  w2 = W_w.T
  b2 = W_b.reshape(1, OUT)
  return pl.pallas_call(
      _mlp_body,
      out_shape=jax.ShapeDtypeStruct((B, OUT), jnp.float32),
  )(sums, w1, b1, w2, b2)


# speedup vs baseline: 2.1309x; 1.7482x over previous
import functools

import jax
import jax.numpy as jnp
from jax import lax
from jax.experimental import pallas as pl
from jax.experimental.pallas import tpu as pltpu
from jax.experimental.pallas import tpu_sc as plsc

VOCAB = 100000
D = 300
B = 1024
L = 200
HID = 32
OUT = 2

NC = 2
NS = 16
NW = NC * NS
RPW = B // NW
G = 40
NCHUNK = L // G
DP = 304
NACC = DP // 16


def _sc_sums(x, table_pad):
  mesh = plsc.VectorSubcoreMesh(core_axis_name="c", subcore_axis_name="s")

  @functools.partial(
      pl.kernel,
      out_type=jax.ShapeDtypeStruct((B, DP), jnp.float32),
      mesh=mesh,
      compiler_params=pltpu.CompilerParams(
          use_tc_tiling_on_sc=False, needs_layout_passes=False),
      scratch_types=(
          [pltpu.VMEM((RPW, L), jnp.int32)]
          + [pltpu.VMEM((G, DP), jnp.float32) for _ in range(NCHUNK)]
          + [pltpu.VMEM((RPW, DP), jnp.float32)]
          + [pltpu.SemaphoreType.DMA for _ in range(NCHUNK)]
      ),
  )
  def k(x_hbm, tab_hbm, out_hbm, idx_v, *rest):
    bufs = rest[:NCHUNK]
    ostage = rest[NCHUNK]
    sems = rest[NCHUNK + 1:]
    wid = lax.axis_index("s") * NC + lax.axis_index("c")
    base = wid * RPW
    pltpu.sync_copy(x_hbm.at[pl.ds(base, RPW)], idx_v)

    for c in range(NCHUNK):
      pltpu.async_copy(tab_hbm.at[idx_v.at[0, pl.ds(c * G, G)]],
                       bufs[c], sems[c])

    @pl.loop(0, RPW)
    def _row(i):
      accs = tuple(jnp.zeros((16,), jnp.float32) for _ in range(NACC))
      for c in range(NCHUNK):
        pltpu.make_async_copy(tab_hbm.at[idx_v.at[i, pl.ds(c * G, G)]],
                              bufs[c], sems[c]).wait()

        def body(r, a, _buf=bufs[c]):
          return tuple(x + _buf[r, pl.ds(kk * 16, 16)]
                       for kk, x in enumerate(a))

        accs = lax.fori_loop(0, G, body, accs)

        @pl.when(i + 1 < RPW)
        def _():
          pltpu.async_copy(tab_hbm.at[idx_v.at[i + 1, pl.ds(c * G, G)]],
                           bufs[c], sems[c])

      for kk in range(NACC):
        ostage[i, pl.ds(kk * 16, 16)] = accs[kk]

    pltpu.sync_copy(ostage, out_hbm.at[pl.ds(base, RPW)])

  return k(x, table_pad)


PAD_ROWS = 10000


def _pad_body(t_ref, o_ref):
  o_ref[:, pl.ds(0, D)] = t_ref[...]


def _pad_table(emb_table):
  return pl.pallas_call(
      _pad_body,
      grid=(VOCAB // PAD_ROWS,),
      in_specs=[pl.BlockSpec((PAD_ROWS, D), lambda i: (i, 0))],
      out_specs=pl.BlockSpec((PAD_ROWS, DP), lambda i: (i, 0)),
      out_shape=jax.ShapeDtypeStruct((VOCAB, DP), jnp.float32),
  )(emb_table)


def _mlp_body(s_ref, w1_ref, b1_ref, w2_ref, b2_ref, o_ref):
  h = jnp.dot(s_ref[:, pl.ds(0, D)], w1_ref[...],
              preferred_element_type=jnp.float32)
  h = jnp.maximum(h + b1_ref[...], 0.0)
  logits = jnp.dot(h, w2_ref[...], preferred_element_type=jnp.float32)
  logits = logits + b2_ref[...]
  m = jnp.max(logits, axis=0, keepdims=True)
  z = logits - m
  o_ref[...] = z - jnp.log(jnp.sum(jnp.exp(z), axis=0, keepdims=True))


def kernel(x, emb_table, V_w, V_b, W_w, W_b):
  x = x.astype(jnp.int32)
  table_pad = _pad_table(emb_table)
  sums = _sc_sums(x, table_pad)
  w1 = V_w.T * (1.0 / L)
  b1 = V_b.reshape(1, HID)
  w2 = W_w.T
  b2 = W_b.reshape(1, OUT)
  return pl.pallas_call(
      _mlp_body,
      out_shape=jax.ShapeDtypeStruct((B, OUT), jnp.float32),
  )(sums, w1, b1, w2, b2)
